# baseline (speedup 1.0000x reference)
import functools

import jax
import jax.numpy as jnp
from jax import lax
from jax.experimental import pallas as pl
from jax.experimental.pallas import tpu as pltpu
from jax.experimental.pallas import tpu_sc as plsc

_N = 8192
_K = 20
_EPS = 1e-5
_NEG = -3.0e38
_F32 = jnp.float32
_ACC = 8


def _elu(z):
    zc = jnp.minimum(z, 0.0)
    p = zc * (1.0 + zc / 2.0 * (1.0 + zc / 3.0 * (1.0 + zc / 4.0 *
        (1.0 + zc / 5.0 * (1.0 + zc / 6.0 * (1.0 + zc / 7.0))))))
    e = jnp.where(z > -0.25, p, jnp.exp(zc) - 1.0)
    return jnp.where(z > 0.0, z, e)


def _topk_body(xt_ref, x_ref, idx_ref):
    xt = xt_ref[...]
    xf = x_ref[...]
    g = lax.dot_general(xt, xf, (((1,), (0,)), ((), ())),
                        preferred_element_type=_F32)
    xx_r = jnp.sum(xt * xt, axis=1, keepdims=True)
    xx_c = jnp.sum(xf * xf, axis=0, keepdims=True)
    s = (2.0 * g - xx_r) - xx_c
    col = lax.broadcasted_iota(jnp.int32, s.shape, 1)
    for k in range(_K):
        m = jnp.max(s, axis=1, keepdims=True)
        a = jnp.min(jnp.where(s == m, col, _N), axis=1, keepdims=True)
        idx_ref[:, pl.ds(k, 1)] = a
        if k + 1 < _K:
            s = jnp.where(col == a, _NEG, s)


def _topk(xt, xf, tr):
    n, c = xt.shape
    return pl.pallas_call(
        _topk_body,
        grid=(n // tr,),
        in_specs=[pl.BlockSpec((tr, c), lambda i: (i, 0)),
                  pl.BlockSpec((c, n), lambda i: (0, 0))],
        out_specs=pl.BlockSpec((tr, _K), lambda i: (i, 0)),
        out_shape=jax.ShapeDtypeStruct((n, _K), jnp.int32),
    )(xt, xf)


def _sc_gather(table, idx_flat):
    n, d = table.shape
    info = plsc.get_sparse_core_info()
    ncores = info.num_cores
    nw = ncores * info.num_subcores
    ppw = n // nw
    bp = 4
    ib = bp * _K
    nb = ppw // bp

    def body(tab_hbm, idx_hbm, out_hbm, idx_v, rows_v, sem):
        wid = lax.axis_index("s") * ncores + lax.axis_index("c")

        def batch(bi, carry):
            ebase = (wid * ppw + bi * bp) * _K
            pltpu.sync_copy(idx_hbm.at[pl.ds(ebase, ib)], idx_v)
            pltpu.async_copy(tab_hbm.at[idx_v], rows_v, sem).wait()
            pltpu.sync_copy(rows_v, out_hbm.at[pl.ds(ebase, ib)])
            return carry

        lax.fori_loop(0, nb, batch, 0)

    mesh = plsc.VectorSubcoreMesh(core_axis_name="c", subcore_axis_name="s")
    f = pl.kernel(
        body,
        mesh=mesh,
        out_type=jax.ShapeDtypeStruct((n * _K, d), _F32),
        scratch_types=[
            pltpu.VMEM((ib,), jnp.int32),
            pltpu.VMEM((ib, d), _F32),
            pltpu.SemaphoreType.DMA,
        ],
    )
    return f(table, idx_flat)


def _conv_y(cin, xg_ref, xtb, w, j):
    xn = xg_ref[:, j, :cin]
    f = jnp.concatenate([xn - xtb, xtb], axis=1)
    return lax.dot_general(f, w, (((1,), (1,)), ((), ())),
                           preferred_element_type=_F32)


def _conv3_body(cin, xg_ref, xt_ref, w_ref, y_ref, mmax_ref, mmin_ref):
    xtb = xt_ref[...]
    w = w_ref[...]
    m_mx = None
    for j in range(_K):
        y = _conv_y(cin, xg_ref, xtb, w, j)
        y_ref[:, j, :] = y
        if m_mx is None:
            m_mx = y
            m_mn = y
        else:
            m_mx = jnp.maximum(m_mx, y)
            m_mn = jnp.minimum(m_mn, y)
    mmax_ref[...] = m_mx
    mmin_ref[...] = m_mn


def _conv3(xg, xt, w, tr):
    n, c = xt.shape
    d = xg.shape[-1]
    cout = w.shape[0]
    return pl.pallas_call(
        functools.partial(_conv3_body, c),
        grid=(n // tr,),
        in_specs=[pl.BlockSpec((tr, _K, d), lambda i: (i, 0, 0)),
                  pl.BlockSpec((tr, c), lambda i: (i, 0)),
                  pl.BlockSpec((cout, 2 * c), lambda i: (0, 0))],
        out_specs=[pl.BlockSpec((tr, _K, cout), lambda i: (i, 0, 0)),
                   pl.BlockSpec((tr, cout), lambda i: (i, 0)),
                   pl.BlockSpec((tr, cout), lambda i: (i, 0))],
        out_shape=[jax.ShapeDtypeStruct((n, _K, cout), _F32),
                   jax.ShapeDtypeStruct((n, cout), _F32),
                   jax.ShapeDtypeStruct((n, cout), _F32)],
    )(xg, xt, w)


def _conv1_body(cin, xg_ref, xt_ref, w_ref, acc_ref):
    xtb = xt_ref[...]
    w = w_ref[...]
    s1 = None
    for j in range(_K):
        y = _conv_y(cin, xg_ref, xtb, w, j)
        c = jnp.sum(y, axis=0, keepdims=True)
        s1 = c if s1 is None else s1 + c
    i = pl.program_id(0)

    @pl.when(i == 0)
    def _():
        acc_ref[...] = jnp.zeros_like(acc_ref)

    acc_ref[pl.ds(jnp.remainder(i, _ACC), 1), :] += s1


def _conv1(xg, xt, w, tr):
    n, c = xt.shape
    d = xg.shape[-1]
    cout = w.shape[0]
    return pl.pallas_call(
        functools.partial(_conv1_body, c),
        grid=(n // tr,),
        in_specs=[pl.BlockSpec((tr, _K, d), lambda i: (i, 0, 0)),
                  pl.BlockSpec((tr, c), lambda i: (i, 0)),
                  pl.BlockSpec((cout, 2 * c), lambda i: (0, 0))],
        out_specs=pl.BlockSpec((_ACC, cout), lambda i: (0, 0)),
        out_shape=jax.ShapeDtypeStruct((_ACC, cout), _F32),
    )(xg, xt, w)


def _conv2_body(cin, cnt, xg_ref, xt_ref, w_ref, acc1_ref,
                mmax_ref, mmin_ref, acc_ref):
    xtb = xt_ref[...]
    w = w_ref[...]
    mu = jnp.sum(acc1_ref[...], axis=0, keepdims=True) / cnt
    m_mx = None
    s2 = None
    for j in range(_K):
        y = _conv_y(cin, xg_ref, xtb, w, j)
        d = y - mu
        c = jnp.sum(d * d, axis=0, keepdims=True)
        if m_mx is None:
            m_mx = y
            m_mn = y
            s2 = c
        else:
            m_mx = jnp.maximum(m_mx, y)
            m_mn = jnp.minimum(m_mn, y)
            s2 = s2 + c
    mmax_ref[...] = m_mx
    mmin_ref[...] = m_mn
    i = pl.program_id(0)

    @pl.when(i == 0)
    def _():
        acc_ref[...] = jnp.zeros_like(acc_ref)

    acc_ref[pl.ds(jnp.remainder(i, _ACC), 1), :] += s2


def _conv2(xg, xt, w, acc1, tr):
    n, c = xt.shape
    d = xg.shape[-1]
    cout = w.shape[0]
    return pl.pallas_call(
        functools.partial(_conv2_body, c, float(_N * _K)),
        grid=(n // tr,),
        in_specs=[pl.BlockSpec((tr, _K, d), lambda i: (i, 0, 0)),
                  pl.BlockSpec((tr, c), lambda i: (i, 0)),
                  pl.BlockSpec((cout, 2 * c), lambda i: (0, 0)),
                  pl.BlockSpec((_ACC, cout), lambda i: (0, 0))],
        out_specs=[pl.BlockSpec((tr, cout), lambda i: (i, 0)),
                   pl.BlockSpec((tr, cout), lambda i: (i, 0)),
                   pl.BlockSpec((_ACC, cout), lambda i: (0, 0))],
        out_shape=[jax.ShapeDtypeStruct((n, cout), _F32),
                   jax.ShapeDtypeStruct((n, cout), _F32),
                   jax.ShapeDtypeStruct((_ACC, cout), _F32)],
    )(xg, xt, w, acc1)


def _apply_body(mmax_ref, mmin_ref, mu_ref, var_ref, g_ref, b_ref, out_ref):
    mu = mu_ref[...]
    sq = jnp.sqrt(var_ref[...] + _EPS)
    g = g_ref[...]
    m = jnp.where(g >= 0.0, mmax_ref[...], mmin_ref[...])
    out_ref[...] = _elu(((m - mu) / sq) * g + b_ref[...])


def _apply(mmax, mmin, mu, var, g, b, tr):
    n, c = mmax.shape
    return pl.pallas_call(
        _apply_body,
        grid=(n // tr,),
        in_specs=[pl.BlockSpec((tr, c), lambda i: (i, 0))] * 2
        + [pl.BlockSpec((1, c), lambda i: (0, 0))] * 4,
        out_specs=pl.BlockSpec((tr, c), lambda i: (i, 0)),
        out_shape=jax.ShapeDtypeStruct((n, c), _F32),
    )(mmax, mmin, mu, var, g, b)


def _apply4_body(cnt, mmax_ref, mmin_ref, acc1_ref, acc2_ref,
                 g_ref, b_ref, out_ref):
    mu = jnp.sum(acc1_ref[...], axis=0, keepdims=True) / cnt
    var = jnp.sum(acc2_ref[...], axis=0, keepdims=True) / cnt
    sq = jnp.sqrt(var + _EPS)
    g = g_ref[...]
    m = jnp.where(g >= 0.0, mmax_ref[...], mmin_ref[...])
    out_ref[...] = _elu(((m - mu) / sq) * g + b_ref[...])


def _apply4(mmax, mmin, acc1, acc2, g, b, cnt, tr):
    n, c = mmax.shape
    return pl.pallas_call(
        functools.partial(_apply4_body, cnt),
        grid=(n // tr,),
        in_specs=[pl.BlockSpec((tr, c), lambda i: (i, 0))] * 2
        + [pl.BlockSpec((_ACC, c), lambda i: (0, 0))] * 2
        + [pl.BlockSpec((1, c), lambda i: (0, 0))] * 2,
        out_specs=pl.BlockSpec((tr, c), lambda i: (i, 0)),
        out_shape=jax.ShapeDtypeStruct((n, c), _F32),
    )(mmax, mmin, acc1, acc2, g, b)


def _fc_body(cat_ref, w_ref, b_ref, h_ref, acc_ref):
    h = lax.dot_general(cat_ref[...], w_ref[...], (((1,), (1,)), ((), ())),
                        preferred_element_type=_F32)
    h = h + b_ref[...]
    h_ref[...] = h
    s1 = jnp.sum(h, axis=0, keepdims=True)
    i = pl.program_id(0)

    @pl.when(i == 0)
    def _():
        acc_ref[...] = jnp.zeros_like(acc_ref)

    acc_ref[pl.ds(jnp.remainder(i, _ACC), 1), :] += s1


def _fc(cat, w, b, tr):
    n, cin = cat.shape
    cout = w.shape[0]
    return pl.pallas_call(
        _fc_body,
        grid=(n // tr,),
        in_specs=[pl.BlockSpec((tr, cin), lambda i: (i, 0)),
                  pl.BlockSpec((cout, cin), lambda i: (0, 0)),
                  pl.BlockSpec((1, cout), lambda i: (0, 0))],
        out_specs=[pl.BlockSpec((tr, cout), lambda i: (i, 0)),
                   pl.BlockSpec((_ACC, cout), lambda i: (0, 0))],
        out_shape=[jax.ShapeDtypeStruct((n, cout), _F32),
                   jax.ShapeDtypeStruct((_ACC, cout), _F32)],
    )(cat, w, b)


def _hvar_body(cnt, h_ref, acc1_ref, acc_ref):
    mu = jnp.sum(acc1_ref[...], axis=0, keepdims=True) / cnt
    d = h_ref[...] - mu
    s2 = jnp.sum(d * d, axis=0, keepdims=True)
    i = pl.program_id(0)

    @pl.when(i == 0)
    def _():
        acc_ref[...] = jnp.zeros_like(acc_ref)

    acc_ref[pl.ds(jnp.remainder(i, _ACC), 1), :] += s2


def _hvar(h, acc1, tr):
    n, c = h.shape
    return pl.pallas_call(
        functools.partial(_hvar_body, float(_N)),
        grid=(n // tr,),
        in_specs=[pl.BlockSpec((tr, c), lambda i: (i, 0)),
                  pl.BlockSpec((_ACC, c), lambda i: (0, 0))],
        out_specs=pl.BlockSpec((_ACC, c), lambda i: (0, 0)),
        out_shape=jax.ShapeDtypeStruct((_ACC, c), _F32),
    )(h, acc1)


def _happly_body(cnt, h_ref, acc1_ref, acc2_ref, g_ref, b_ref, out_ref):
    mu = jnp.sum(acc1_ref[...], axis=0, keepdims=True) / cnt
    var = jnp.sum(acc2_ref[...], axis=0, keepdims=True) / cnt
    sq = jnp.sqrt(var + _EPS)
    z = ((h_ref[...] - mu) / sq) * g_ref[...] + b_ref[...]
    out_ref[...] = _elu(z)


def _happly(h, acc1, acc2, g, b, tr):
    n, c = h.shape
    return pl.pallas_call(
        functools.partial(_happly_body, float(_N)),
        grid=(n // tr,),
        in_specs=[pl.BlockSpec((tr, c), lambda i: (i, 0))]
        + [pl.BlockSpec((_ACC, c), lambda i: (0, 0))] * 2
        + [pl.BlockSpec((1, c), lambda i: (0, 0)),
           pl.BlockSpec((1, c), lambda i: (0, 0))],
        out_specs=pl.BlockSpec((tr, c), lambda i: (i, 0)),
        out_shape=jax.ShapeDtypeStruct((n, c), _F32),
    )(h, acc1, acc2, g, b)


def _edge_block(xt, w, g, b):
    n, c = xt.shape
    idx = _topk(xt, xt.T, 256)
    table = jnp.pad(xt, ((0, 0), (0, 128 - c))) if c < 128 else xt
    xg = _sc_gather(table, idx.reshape(-1)).reshape(n, _K, -1)
    y, mmax, mmin = _conv3(xg, xt, w, 512)
    yt = y.transpose(2, 0, 1)
    mu = jnp.mean(yt, axis=(1, 2))
    var = jnp.var(yt, axis=(1, 2))
    m = jnp.where(g >= 0.0, mmax, mmin)
    yn = (m - mu[None, :]) / jnp.sqrt(var + _EPS)[None, :]
    return jax.nn.elu(yn * g[None, :] + b[None, :])


def _edge_block4(xt, w, g, b):
    n, c = xt.shape
    idx = _topk(xt, xt.T, 256)
    xg = _sc_gather(xt, idx.reshape(-1)).reshape(n, _K, -1)
    acc1 = _conv1(xg, xt, w, 512)
    mmax, mmin, acc2 = _conv2(xg, xt, w, acc1, 512)
    return _apply4(mmax, mmin, acc1, acc2, g.reshape(1, -1), b.reshape(1, -1),
                   float(_N * _K), 512)


def kernel(x, W1, g1, b1, W2, g2, b2, W3, g3, b3, W4, g4, b4,
           fcw, fcb, g5, b5):
    x1 = _edge_block(x, W1, g1, b1)
    x2 = _edge_block(x1, W2, g2, b2)
    x3 = _edge_block(x2, W3, g3, b3)
    x4 = _edge_block4(x3, W4, g4, b4)
    cat = jnp.concatenate([x1, x2, x3, x4], axis=1)
    h, acc1 = _fc(cat, fcw, fcb.reshape(1, -1), 512)
    acc2 = _hvar(h, acc1, 512)
    return _happly(h, acc1, acc2, g5.reshape(1, -1), b5.reshape(1, -1), 512)


# speedup vs baseline: 1.2206x; 1.2206x over previous
import functools

import jax
import jax.numpy as jnp
from jax import lax
from jax.experimental import pallas as pl
from jax.experimental.pallas import tpu as pltpu
from jax.experimental.pallas import tpu_sc as plsc

_N = 8192
_K = 20
_EPS = 1e-5
_NEG = -3.0e38
_F32 = jnp.float32
_ACC = 8


def _elu(z):
    zc = jnp.minimum(z, 0.0)
    p = zc * (1.0 + zc / 2.0 * (1.0 + zc / 3.0 * (1.0 + zc / 4.0 *
        (1.0 + zc / 5.0 * (1.0 + zc / 6.0 * (1.0 + zc / 7.0))))))
    e = jnp.where(z > -0.25, p, jnp.exp(zc) - 1.0)
    return jnp.where(z > 0.0, z, e)


def _topk_body(xt_ref, x_ref, idx_ref):
    xt = xt_ref[...]
    xf = x_ref[...]
    g = lax.dot_general(xt, xf, (((1,), (0,)), ((), ())),
                        preferred_element_type=_F32)
    xx_r = jnp.sum(xt * xt, axis=1, keepdims=True)
    xx_c = jnp.sum(xf * xf, axis=0, keepdims=True)
    s = (2.0 * g - xx_r) - xx_c
    col = lax.broadcasted_iota(jnp.int32, s.shape, 1)
    for k in range(_K):
        a = jnp.argmax(s, axis=1).astype(jnp.int32)[:, None]
        idx_ref[:, pl.ds(k, 1)] = a
        if k + 1 < _K:
            s = jnp.where(col == a, _NEG, s)


def _topk(xt, xf, tr):
    n, c = xt.shape
    return pl.pallas_call(
        _topk_body,
        grid=(n // tr,),
        in_specs=[pl.BlockSpec((tr, c), lambda i: (i, 0)),
                  pl.BlockSpec((c, n), lambda i: (0, 0))],
        out_specs=pl.BlockSpec((tr, _K), lambda i: (i, 0)),
        out_shape=jax.ShapeDtypeStruct((n, _K), jnp.int32),
    )(xt, xf)


def _sc_gather(table, idx_flat):
    n, d = table.shape
    info = plsc.get_sparse_core_info()
    ncores = info.num_cores
    nw = ncores * info.num_subcores
    ppw = n // nw
    bp = 4
    ib = bp * _K
    nb = ppw // bp

    def body(tab_hbm, idx_hbm, out_hbm, idx_v, rows_v, sem):
        wid = lax.axis_index("s") * ncores + lax.axis_index("c")

        def batch(bi, carry):
            ebase = (wid * ppw + bi * bp) * _K
            pltpu.sync_copy(idx_hbm.at[pl.ds(ebase, ib)], idx_v)
            pltpu.async_copy(tab_hbm.at[idx_v], rows_v, sem).wait()
            pltpu.sync_copy(rows_v, out_hbm.at[pl.ds(ebase, ib)])
            return carry

        lax.fori_loop(0, nb, batch, 0)

    mesh = plsc.VectorSubcoreMesh(core_axis_name="c", subcore_axis_name="s")
    f = pl.kernel(
        body,
        mesh=mesh,
        out_type=jax.ShapeDtypeStruct((n * _K, d), _F32),
        scratch_types=[
            pltpu.VMEM((ib,), jnp.int32),
            pltpu.VMEM((ib, d), _F32),
            pltpu.SemaphoreType.DMA,
        ],
    )
    return f(table, idx_flat)


def _conv_y(cin, xg_ref, xtb, w, j):
    xn = xg_ref[:, j, :cin]
    f = jnp.concatenate([xn - xtb, xtb], axis=1)
    return lax.dot_general(f, w, (((1,), (1,)), ((), ())),
                           preferred_element_type=_F32)


def _conv3_body(cin, xg_ref, xt_ref, w_ref, y_ref, mmax_ref, mmin_ref):
    xtb = xt_ref[...]
    w = w_ref[...]
    m_mx = None
    for j in range(_K):
        y = _conv_y(cin, xg_ref, xtb, w, j)
        y_ref[:, j, :] = y
        if m_mx is None:
            m_mx = y
            m_mn = y
        else:
            m_mx = jnp.maximum(m_mx, y)
            m_mn = jnp.minimum(m_mn, y)
    mmax_ref[...] = m_mx
    mmin_ref[...] = m_mn


def _conv3(xg, xt, w, tr):
    n, c = xt.shape
    d = xg.shape[-1]
    cout = w.shape[0]
    return pl.pallas_call(
        functools.partial(_conv3_body, c),
        grid=(n // tr,),
        in_specs=[pl.BlockSpec((tr, _K, d), lambda i: (i, 0, 0)),
                  pl.BlockSpec((tr, c), lambda i: (i, 0)),
                  pl.BlockSpec((cout, 2 * c), lambda i: (0, 0))],
        out_specs=[pl.BlockSpec((tr, _K, cout), lambda i: (i, 0, 0)),
                   pl.BlockSpec((tr, cout), lambda i: (i, 0)),
                   pl.BlockSpec((tr, cout), lambda i: (i, 0))],
        out_shape=[jax.ShapeDtypeStruct((n, _K, cout), _F32),
                   jax.ShapeDtypeStruct((n, cout), _F32),
                   jax.ShapeDtypeStruct((n, cout), _F32)],
    )(xg, xt, w)


def _conv1_body(cin, xg_ref, xt_ref, w_ref, acc_ref):
    xtb = xt_ref[...]
    w = w_ref[...]
    s1 = None
    for j in range(_K):
        y = _conv_y(cin, xg_ref, xtb, w, j)
        c = jnp.sum(y, axis=0, keepdims=True)
        s1 = c if s1 is None else s1 + c
    i = pl.program_id(0)

    @pl.when(i == 0)
    def _():
        acc_ref[...] = jnp.zeros_like(acc_ref)

    acc_ref[pl.ds(jnp.remainder(i, _ACC), 1), :] += s1


def _conv1(xg, xt, w, tr):
    n, c = xt.shape
    d = xg.shape[-1]
    cout = w.shape[0]
    return pl.pallas_call(
        functools.partial(_conv1_body, c),
        grid=(n // tr,),
        in_specs=[pl.BlockSpec((tr, _K, d), lambda i: (i, 0, 0)),
                  pl.BlockSpec((tr, c), lambda i: (i, 0)),
                  pl.BlockSpec((cout, 2 * c), lambda i: (0, 0))],
        out_specs=pl.BlockSpec((_ACC, cout), lambda i: (0, 0)),
        out_shape=jax.ShapeDtypeStruct((_ACC, cout), _F32),
    )(xg, xt, w)


def _conv2_body(cin, cnt, xg_ref, xt_ref, w_ref, acc1_ref,
                mmax_ref, mmin_ref, acc_ref):
    xtb = xt_ref[...]
    w = w_ref[...]
    mu = jnp.sum(acc1_ref[...], axis=0, keepdims=True) / cnt
    m_mx = None
    s2 = None
    for j in range(_K):
        y = _conv_y(cin, xg_ref, xtb, w, j)
        d = y - mu
        c = jnp.sum(d * d, axis=0, keepdims=True)
        if m_mx is None:
            m_mx = y
            m_mn = y
            s2 = c
        else:
            m_mx = jnp.maximum(m_mx, y)
            m_mn = jnp.minimum(m_mn, y)
            s2 = s2 + c
    mmax_ref[...] = m_mx
    mmin_ref[...] = m_mn
    i = pl.program_id(0)

    @pl.when(i == 0)
    def _():
        acc_ref[...] = jnp.zeros_like(acc_ref)

    acc_ref[pl.ds(jnp.remainder(i, _ACC), 1), :] += s2


def _conv2(xg, xt, w, acc1, tr):
    n, c = xt.shape
    d = xg.shape[-1]
    cout = w.shape[0]
    return pl.pallas_call(
        functools.partial(_conv2_body, c, float(_N * _K)),
        grid=(n // tr,),
        in_specs=[pl.BlockSpec((tr, _K, d), lambda i: (i, 0, 0)),
                  pl.BlockSpec((tr, c), lambda i: (i, 0)),
                  pl.BlockSpec((cout, 2 * c), lambda i: (0, 0)),
                  pl.BlockSpec((_ACC, cout), lambda i: (0, 0))],
        out_specs=[pl.BlockSpec((tr, cout), lambda i: (i, 0)),
                   pl.BlockSpec((tr, cout), lambda i: (i, 0)),
                   pl.BlockSpec((_ACC, cout), lambda i: (0, 0))],
        out_shape=[jax.ShapeDtypeStruct((n, cout), _F32),
                   jax.ShapeDtypeStruct((n, cout), _F32),
                   jax.ShapeDtypeStruct((_ACC, cout), _F32)],
    )(xg, xt, w, acc1)


def _apply_body(mmax_ref, mmin_ref, mu_ref, var_ref, g_ref, b_ref, out_ref):
    mu = mu_ref[...]
    sq = jnp.sqrt(var_ref[...] + _EPS)
    g = g_ref[...]
    m = jnp.where(g >= 0.0, mmax_ref[...], mmin_ref[...])
    out_ref[...] = _elu(((m - mu) / sq) * g + b_ref[...])


def _apply(mmax, mmin, mu, var, g, b, tr):
    n, c = mmax.shape
    return pl.pallas_call(
        _apply_body,
        grid=(n // tr,),
        in_specs=[pl.BlockSpec((tr, c), lambda i: (i, 0))] * 2
        + [pl.BlockSpec((1, c), lambda i: (0, 0))] * 4,
        out_specs=pl.BlockSpec((tr, c), lambda i: (i, 0)),
        out_shape=jax.ShapeDtypeStruct((n, c), _F32),
    )(mmax, mmin, mu, var, g, b)


def _apply4_body(cnt, mmax_ref, mmin_ref, acc1_ref, acc2_ref,
                 g_ref, b_ref, out_ref):
    mu = jnp.sum(acc1_ref[...], axis=0, keepdims=True) / cnt
    var = jnp.sum(acc2_ref[...], axis=0, keepdims=True) / cnt
    sq = jnp.sqrt(var + _EPS)
    g = g_ref[...]
    m = jnp.where(g >= 0.0, mmax_ref[...], mmin_ref[...])
    out_ref[...] = _elu(((m - mu) / sq) * g + b_ref[...])


def _apply4(mmax, mmin, acc1, acc2, g, b, cnt, tr):
    n, c = mmax.shape
    return pl.pallas_call(
        functools.partial(_apply4_body, cnt),
        grid=(n // tr,),
        in_specs=[pl.BlockSpec((tr, c), lambda i: (i, 0))] * 2
        + [pl.BlockSpec((_ACC, c), lambda i: (0, 0))] * 2
        + [pl.BlockSpec((1, c), lambda i: (0, 0))] * 2,
        out_specs=pl.BlockSpec((tr, c), lambda i: (i, 0)),
        out_shape=jax.ShapeDtypeStruct((n, c), _F32),
    )(mmax, mmin, acc1, acc2, g, b)


def _fc_body(cat_ref, w_ref, b_ref, h_ref, acc_ref):
    h = lax.dot_general(cat_ref[...], w_ref[...], (((1,), (1,)), ((), ())),
                        preferred_element_type=_F32)
    h = h + b_ref[...]
    h_ref[...] = h
    s1 = jnp.sum(h, axis=0, keepdims=True)
    i = pl.program_id(0)

    @pl.when(i == 0)
    def _():
        acc_ref[...] = jnp.zeros_like(acc_ref)

    acc_ref[pl.ds(jnp.remainder(i, _ACC), 1), :] += s1


def _fc(cat, w, b, tr):
    n, cin = cat.shape
    cout = w.shape[0]
    return pl.pallas_call(
        _fc_body,
        grid=(n // tr,),
        in_specs=[pl.BlockSpec((tr, cin), lambda i: (i, 0)),
                  pl.BlockSpec((cout, cin), lambda i: (0, 0)),
                  pl.BlockSpec((1, cout), lambda i: (0, 0))],
        out_specs=[pl.BlockSpec((tr, cout), lambda i: (i, 0)),
                   pl.BlockSpec((_ACC, cout), lambda i: (0, 0))],
        out_shape=[jax.ShapeDtypeStruct((n, cout), _F32),
                   jax.ShapeDtypeStruct((_ACC, cout), _F32)],
    )(cat, w, b)


def _hvar_body(cnt, h_ref, acc1_ref, acc_ref):
    mu = jnp.sum(acc1_ref[...], axis=0, keepdims=True) / cnt
    d = h_ref[...] - mu
    s2 = jnp.sum(d * d, axis=0, keepdims=True)
    i = pl.program_id(0)

    @pl.when(i == 0)
    def _():
        acc_ref[...] = jnp.zeros_like(acc_ref)

    acc_ref[pl.ds(jnp.remainder(i, _ACC), 1), :] += s2


def _hvar(h, acc1, tr):
    n, c = h.shape
    return pl.pallas_call(
        functools.partial(_hvar_body, float(_N)),
        grid=(n // tr,),
        in_specs=[pl.BlockSpec((tr, c), lambda i: (i, 0)),
                  pl.BlockSpec((_ACC, c), lambda i: (0, 0))],
        out_specs=pl.BlockSpec((_ACC, c), lambda i: (0, 0)),
        out_shape=jax.ShapeDtypeStruct((_ACC, c), _F32),
    )(h, acc1)


def _happly_body(cnt, h_ref, acc1_ref, acc2_ref, g_ref, b_ref, out_ref):
    mu = jnp.sum(acc1_ref[...], axis=0, keepdims=True) / cnt
    var = jnp.sum(acc2_ref[...], axis=0, keepdims=True) / cnt
    sq = jnp.sqrt(var + _EPS)
    z = ((h_ref[...] - mu) / sq) * g_ref[...] + b_ref[...]
    out_ref[...] = _elu(z)


def _happly(h, acc1, acc2, g, b, tr):
    n, c = h.shape
    return pl.pallas_call(
        functools.partial(_happly_body, float(_N)),
        grid=(n // tr,),
        in_specs=[pl.BlockSpec((tr, c), lambda i: (i, 0))]
        + [pl.BlockSpec((_ACC, c), lambda i: (0, 0))] * 2
        + [pl.BlockSpec((1, c), lambda i: (0, 0)),
           pl.BlockSpec((1, c), lambda i: (0, 0))],
        out_specs=pl.BlockSpec((tr, c), lambda i: (i, 0)),
        out_shape=jax.ShapeDtypeStruct((n, c), _F32),
    )(h, acc1, acc2, g, b)


def _edge_block(xt, w, g, b):
    n, c = xt.shape
    idx = _topk(xt, xt.T, 128)
    table = jnp.pad(xt, ((0, 0), (0, 128 - c))) if c < 128 else xt
    xg = _sc_gather(table, idx.reshape(-1)).reshape(n, _K, -1)
    y, mmax, mmin = _conv3(xg, xt, w, 512)
    yt = y.transpose(2, 0, 1)
    mu = jnp.mean(yt, axis=(1, 2))
    var = jnp.var(yt, axis=(1, 2))
    m = jnp.where(g >= 0.0, mmax, mmin)
    yn = (m - mu[None, :]) / jnp.sqrt(var + _EPS)[None, :]
    return jax.nn.elu(yn * g[None, :] + b[None, :])


def _edge_block4(xt, w, g, b):
    n, c = xt.shape
    idx = _topk(xt, xt.T, 128)
    xg = _sc_gather(xt, idx.reshape(-1)).reshape(n, _K, -1)
    acc1 = _conv1(xg, xt, w, 512)
    mmax, mmin, acc2 = _conv2(xg, xt, w, acc1, 512)
    return _apply4(mmax, mmin, acc1, acc2, g.reshape(1, -1), b.reshape(1, -1),
                   float(_N * _K), 512)


def kernel(x, W1, g1, b1, W2, g2, b2, W3, g3, b3, W4, g4, b4,
           fcw, fcb, g5, b5):
    x1 = _edge_block(x, W1, g1, b1)
    x2 = _edge_block(x1, W2, g2, b2)
    x3 = _edge_block(x2, W3, g3, b3)
    x4 = _edge_block4(x3, W4, g4, b4)
    cat = jnp.concatenate([x1, x2, x3, x4], axis=1)
    h, acc1 = _fc(cat, fcw, fcb.reshape(1, -1), 512)
    acc2 = _hvar(h, acc1, 512)
    return _happly(h, acc1, acc2, g5.reshape(1, -1), b5.reshape(1, -1), 512)
